# Initial kernel scaffold; baseline (speedup 1.0000x reference)
import functools

import jax
import jax.numpy as jnp
from jax import lax
from jax.experimental import pallas as pl
from jax.experimental.pallas import tpu as pltpu
from jax.experimental.pallas import tpu_sc as plsc

N = 500000
F = 32
NBIN = 256
HIST = F * NBIN

NTILES = 32
ROWS_PER_TILE = N // NTILES
CHUNK = 625
NCHUNK = ROWS_PER_TILE // CHUNK
GBUF = CHUNK + 7


def _sc_body(x_hbm, g_hbm, h_hbm, pg_hbm, ph_hbm,
             xb0, xb1, gb0, gb1, hb0, hb1, histg, histh, sem0, sem1):
  c = lax.axis_index("c")
  s = lax.axis_index("s")
  wid = s * 2 + c
  base = wid * ROWS_PER_TILE

  zeros16 = jnp.zeros((16,), jnp.float32)

  def zero_body(i, _):
    histg[pl.ds(i * 16, 16)] = zeros16
    histh[pl.ds(i * 16, 16)] = zeros16
    return 0
  lax.fori_loop(0, HIST // 16, zero_body, 0)

  xbufs = (xb0, xb1)
  gbufs = (gb0, gb1)
  hbufs = (hb0, hb1)
  sems = (sem0, sem1)

  def start(ci, slot):
    st = base + ci * CHUNK
    ga = (st // 8) * 8
    goff = st - ga
    hx = pltpu.async_copy(x_hbm.at[pl.ds(st * F, CHUNK * F)], xbufs[slot],
                          sems[slot])
    hg = pltpu.async_copy(g_hbm.at[pl.ds(ga, GBUF)], gbufs[slot], sems[slot])
    hh = pltpu.async_copy(h_hbm.at[pl.ds(ga, GBUF)], hbufs[slot], sems[slot])
    return (hx, hg, hh), goff

  iota16 = lax.iota(jnp.int32, 16)
  feats = (iota16, iota16 + 16)

  def process(slot, goff):
    xb = xbufs[slot]
    gb = gbufs[slot]
    hb = hbufs[slot]

    def row_body(r, _):
      gv = jnp.full((16,), gb[r + goff])
      hv = jnp.full((16,), hb[r + goff])
      p = r * F
      for half in range(2):
        xv = xb[pl.ds(p + half * 16, 16)]
        idx = (xv << 5) + feats[half]
        plsc.addupdate_scatter(histg, [idx], gv)
        plsc.addupdate_scatter(histh, [idx], hv)
      return 0
    lax.fori_loop(0, CHUNK, row_body, 0)

  handles, goff_cur = start(0, 0)
  for ci in range(NCHUNK):
    slot = ci % 2
    for hnd in handles:
      hnd.wait()
    goff_now = goff_cur
    if ci + 1 < NCHUNK:
      handles, goff_cur = start(ci + 1, 1 - slot)
    process(slot, goff_now)

  pltpu.sync_copy(histg, pg_hbm.at[pl.ds(wid * HIST, HIST)])
  pltpu.sync_copy(histh, ph_hbm.at[pl.ds(wid * HIST, HIST)])


def _tc_body(pg_ref, ph_ref, gl_ref, hl_ref):
  hg = jnp.sum(pg_ref[...], axis=0)
  hh = jnp.sum(ph_ref[...], axis=0)
  rows = lax.broadcasted_iota(jnp.int32, (NBIN, NBIN), 0)
  cols = lax.broadcasted_iota(jnp.int32, (NBIN, NBIN), 1)
  tri = (rows <= cols).astype(jnp.float32)
  dn = (((0,), (0,)), ((), ()))
  gl_ref[...] = lax.dot_general(hg, tri, dn,
                                preferred_element_type=jnp.float32)
  hl_ref[...] = lax.dot_general(hh, tri, dn,
                                preferred_element_type=jnp.float32)


@jax.jit
def kernel(X, gradient, hessian):
  mesh = plsc.VectorSubcoreMesh(core_axis_name="c", subcore_axis_name="s")
  sc = pl.kernel(
      _sc_body,
      out_type=(
          jax.ShapeDtypeStruct((NTILES * HIST,), jnp.float32),
          jax.ShapeDtypeStruct((NTILES * HIST,), jnp.float32),
      ),
      mesh=mesh,
      scratch_types=[
          pltpu.VMEM((CHUNK * F,), jnp.int32),
          pltpu.VMEM((CHUNK * F,), jnp.int32),
          pltpu.VMEM((GBUF,), jnp.float32),
          pltpu.VMEM((GBUF,), jnp.float32),
          pltpu.VMEM((GBUF,), jnp.float32),
          pltpu.VMEM((GBUF,), jnp.float32),
          pltpu.VMEM((HIST,), jnp.float32),
          pltpu.VMEM((HIST,), jnp.float32),
          pltpu.SemaphoreType.DMA,
          pltpu.SemaphoreType.DMA,
      ],
  )
  pg, ph = sc(X.reshape(-1), gradient, hessian)

  pg3 = pg.reshape(NTILES, NBIN, F)
  ph3 = ph.reshape(NTILES, NBIN, F)
  gl, hl = pl.pallas_call(
      _tc_body,
      out_shape=(
          jax.ShapeDtypeStruct((F, NBIN), jnp.float32),
          jax.ShapeDtypeStruct((F, NBIN), jnp.float32),
      ),
  )(pg3, ph3)
  return (gl[None], hl[None])


# speedup vs baseline: 70.9235x; 70.9235x over previous
import functools

import jax
import jax.numpy as jnp
from jax import lax
from jax.experimental import pallas as pl
from jax.experimental.pallas import tpu as pltpu
from jax.experimental.pallas import tpu_sc as plsc

N = 500000
F = 32
NBIN = 256
HIST = F * NBIN

NTILES = 32
ROWS_PER_TILE = N // NTILES
CHUNK = 625
NCHUNK = ROWS_PER_TILE // CHUNK
GCOPY = CHUNK + 7
GBUF = GCOPY + 16


def _sc_body(x_hbm, g_hbm, h_hbm, pg_hbm, ph_hbm,
             xb0, xb1, gb0, gb1, hb0, hb1, histg, histh, sem0, sem1):
  c = lax.axis_index("c")
  s = lax.axis_index("s")
  wid = s * 2 + c
  base = wid * ROWS_PER_TILE

  zeros16 = jnp.zeros((16,), jnp.float32)

  def zero_body(i, _):
    histg[pl.ds(i * 16, 16)] = zeros16
    histh[pl.ds(i * 16, 16)] = zeros16
    return 0
  lax.fori_loop(0, HIST // 16, zero_body, 0)

  xbufs = (xb0, xb1)
  gbufs = (gb0, gb1)
  hbufs = (hb0, hb1)
  sems = (sem0, sem1)

  def start(ci, slot):
    st = base + ci * CHUNK
    ga = (st // 8) * 8
    goff = st - ga
    hx = pltpu.async_copy(x_hbm.at[pl.ds(st * F, CHUNK * F)], xbufs[slot],
                          sems[slot])
    hg = pltpu.async_copy(g_hbm.at[pl.ds(ga, GCOPY)],
                          gbufs[slot].at[pl.ds(0, GCOPY)], sems[slot])
    hh = pltpu.async_copy(h_hbm.at[pl.ds(ga, GCOPY)],
                          hbufs[slot].at[pl.ds(0, GCOPY)], sems[slot])
    return (hx, hg, hh), goff

  iota16 = lax.iota(jnp.int32, 16)
  feats = (iota16, iota16 + 16)

  def process(slot, goff):
    xb = xbufs[slot]
    gb = gbufs[slot]
    hb = hbufs[slot]

    def row_body(r, _):
      gv = jnp.full((16,), gb[pl.ds(r + goff, 16)][0])
      hv = jnp.full((16,), hb[pl.ds(r + goff, 16)][0])
      p = r * F
      for half in range(2):
        xv = xb[pl.ds(p + half * 16, 16)]
        idx = (xv << 5) + feats[half]
        plsc.addupdate_scatter(histg, [idx], gv)
        plsc.addupdate_scatter(histh, [idx], hv)
      return 0
    lax.fori_loop(0, CHUNK, row_body, 0)

  handles, goff_cur = start(0, 0)
  for ci in range(NCHUNK):
    slot = ci % 2
    for hnd in handles:
      hnd.wait()
    goff_now = goff_cur
    if ci + 1 < NCHUNK:
      handles, goff_cur = start(ci + 1, 1 - slot)
    process(slot, goff_now)

  pltpu.sync_copy(histg, pg_hbm.at[pl.ds(wid * HIST, HIST)])
  pltpu.sync_copy(histh, ph_hbm.at[pl.ds(wid * HIST, HIST)])


def _tc_body(pg_ref, ph_ref, gl_ref, hl_ref):
  hg = jnp.sum(pg_ref[...], axis=0)
  hh = jnp.sum(ph_ref[...], axis=0)
  rows = lax.broadcasted_iota(jnp.int32, (NBIN, NBIN), 0)
  cols = lax.broadcasted_iota(jnp.int32, (NBIN, NBIN), 1)
  tri = (rows <= cols).astype(jnp.float32)
  dn = (((0,), (0,)), ((), ()))
  gl_ref[...] = lax.dot_general(hg, tri, dn,
                                preferred_element_type=jnp.float32)
  hl_ref[...] = lax.dot_general(hh, tri, dn,
                                preferred_element_type=jnp.float32)


@jax.jit
def kernel(X, gradient, hessian):
  mesh = plsc.VectorSubcoreMesh(core_axis_name="c", subcore_axis_name="s")
  sc = pl.kernel(
      _sc_body,
      out_type=(
          jax.ShapeDtypeStruct((NTILES * HIST,), jnp.float32),
          jax.ShapeDtypeStruct((NTILES * HIST,), jnp.float32),
      ),
      mesh=mesh,
      compiler_params=pltpu.CompilerParams(needs_layout_passes=False),
      scratch_types=[
          pltpu.VMEM((CHUNK * F,), jnp.int32),
          pltpu.VMEM((CHUNK * F,), jnp.int32),
          pltpu.VMEM((GBUF,), jnp.float32),
          pltpu.VMEM((GBUF,), jnp.float32),
          pltpu.VMEM((GBUF,), jnp.float32),
          pltpu.VMEM((GBUF,), jnp.float32),
          pltpu.VMEM((HIST,), jnp.float32),
          pltpu.VMEM((HIST,), jnp.float32),
          pltpu.SemaphoreType.DMA,
          pltpu.SemaphoreType.DMA,
      ],
  )
  pg, ph = sc(X.reshape(-1), gradient, hessian)

  pg3 = pg.reshape(NTILES, NBIN, F)
  ph3 = ph.reshape(NTILES, NBIN, F)
  gl, hl = pl.pallas_call(
      _tc_body,
      out_shape=(
          jax.ShapeDtypeStruct((F, NBIN), jnp.float32),
          jax.ShapeDtypeStruct((F, NBIN), jnp.float32),
      ),
  )(pg3, ph3)
  return (gl[None], hl[None])


# speedup vs baseline: 103.5118x; 1.4595x over previous
import functools

import jax
import jax.numpy as jnp
from jax import lax
from jax.experimental import pallas as pl
from jax.experimental.pallas import tpu as pltpu
from jax.experimental.pallas import tpu_sc as plsc

N = 500000
F = 32
NBIN = 256
HIST = F * NBIN

NTILES = 32
ROWS_PER_TILE = N // NTILES
CHUNK = 625
NCHUNK = ROWS_PER_TILE // CHUNK
GCOPY = CHUNK + 7
GBUF = GCOPY + 16


def _sc_body(x_hbm, g_hbm, h_hbm, pg_hbm, ph_hbm,
             xb0, xb1, gb0, gb1, hb0, hb1, histg, histh, sem0, sem1):
  c = lax.axis_index("c")
  s = lax.axis_index("s")
  wid = s * 2 + c
  base = wid * ROWS_PER_TILE

  zeros16 = jnp.zeros((16,), jnp.float32)

  def zero_body(i, _):
    histg[pl.ds(i * 16, 16)] = zeros16
    histh[pl.ds(i * 16, 16)] = zeros16
    return 0
  lax.fori_loop(0, HIST // 16, zero_body, 0)

  xbufs = (xb0, xb1)
  gbufs = (gb0, gb1)
  hbufs = (hb0, hb1)
  sems = (sem0, sem1)

  def start(ci, slot):
    st = base + ci * CHUNK
    ga = (st // 8) * 8
    goff = st - ga
    hx = pltpu.async_copy(x_hbm.at[pl.ds(st * F, CHUNK * F)], xbufs[slot],
                          sems[slot])
    hg = pltpu.async_copy(g_hbm.at[pl.ds(ga, GCOPY)],
                          gbufs[slot].at[pl.ds(0, GCOPY)], sems[slot])
    hh = pltpu.async_copy(h_hbm.at[pl.ds(ga, GCOPY)],
                          hbufs[slot].at[pl.ds(0, GCOPY)], sems[slot])
    return (hx, hg, hh), goff

  iota16 = lax.iota(jnp.int32, 16)
  feats = (iota16, iota16 + 16)

  def process(slot, goff):
    xb = xbufs[slot]
    gb = gbufs[slot]
    hb = hbufs[slot]

    @plsc.parallel_loop(0, CHUNK, 1, unroll=8)
    def row_body(r):
      gv = jnp.full((16,), gb[pl.ds(r + goff, 16)][0])
      hv = jnp.full((16,), hb[pl.ds(r + goff, 16)][0])
      p = r * F
      for half in range(2):
        xv = xb[pl.ds(p + half * 16, 16)]
        idx = (xv << 5) + feats[half]
        plsc.addupdate_scatter(histg, [idx], gv)
        plsc.addupdate_scatter(histh, [idx], hv)

  handles, goff_cur = start(0, 0)
  for ci in range(NCHUNK):
    slot = ci % 2
    for hnd in handles:
      hnd.wait()
    goff_now = goff_cur
    if ci + 1 < NCHUNK:
      handles, goff_cur = start(ci + 1, 1 - slot)
    process(slot, goff_now)

  pltpu.sync_copy(histg, pg_hbm.at[pl.ds(wid * HIST, HIST)])
  pltpu.sync_copy(histh, ph_hbm.at[pl.ds(wid * HIST, HIST)])


def _tc_body(pg_ref, ph_ref, gl_ref, hl_ref):
  hg = jnp.sum(pg_ref[...], axis=0)
  hh = jnp.sum(ph_ref[...], axis=0)
  rows = lax.broadcasted_iota(jnp.int32, (NBIN, NBIN), 0)
  cols = lax.broadcasted_iota(jnp.int32, (NBIN, NBIN), 1)
  tri = (rows <= cols).astype(jnp.float32)
  dn = (((0,), (0,)), ((), ()))
  gl_ref[...] = lax.dot_general(hg, tri, dn,
                                preferred_element_type=jnp.float32)
  hl_ref[...] = lax.dot_general(hh, tri, dn,
                                preferred_element_type=jnp.float32)


@jax.jit
def kernel(X, gradient, hessian):
  mesh = plsc.VectorSubcoreMesh(core_axis_name="c", subcore_axis_name="s")
  sc = pl.kernel(
      _sc_body,
      out_type=(
          jax.ShapeDtypeStruct((NTILES * HIST,), jnp.float32),
          jax.ShapeDtypeStruct((NTILES * HIST,), jnp.float32),
      ),
      mesh=mesh,
      compiler_params=pltpu.CompilerParams(needs_layout_passes=False),
      scratch_types=[
          pltpu.VMEM((CHUNK * F,), jnp.int32),
          pltpu.VMEM((CHUNK * F,), jnp.int32),
          pltpu.VMEM((GBUF,), jnp.float32),
          pltpu.VMEM((GBUF,), jnp.float32),
          pltpu.VMEM((GBUF,), jnp.float32),
          pltpu.VMEM((GBUF,), jnp.float32),
          pltpu.VMEM((HIST,), jnp.float32),
          pltpu.VMEM((HIST,), jnp.float32),
          pltpu.SemaphoreType.DMA,
          pltpu.SemaphoreType.DMA,
      ],
  )
  pg, ph = sc(X.reshape(-1), gradient, hessian)

  pg3 = pg.reshape(NTILES, NBIN, F)
  ph3 = ph.reshape(NTILES, NBIN, F)
  gl, hl = pl.pallas_call(
      _tc_body,
      out_shape=(
          jax.ShapeDtypeStruct((F, NBIN), jnp.float32),
          jax.ShapeDtypeStruct((F, NBIN), jnp.float32),
      ),
  )(pg3, ph3)
  return (gl[None], hl[None])


# speedup vs baseline: 131.2918x; 1.2684x over previous
import jax
import jax.numpy as jnp
from jax import lax
from jax.experimental import pallas as pl
from jax.experimental.pallas import tpu as pltpu
from jax.experimental.pallas import tpu_sc as plsc

N = 500000
F = 32
NBIN = 256
HIST = F * NBIN

NTILES = 32
ROWS_PER_TILE = 15624
CHUNK = 248
NBUF = 3
NOUTER = 21
NCHUNK = NOUTER * NBUF
REM = N - NTILES * ROWS_PER_TILE
GBUF = CHUNK + 16


def _sc_body(x_hbm, g_hbm, h_hbm, pg_hbm, ph_hbm,
             xb0, xb1, xb2, gb0, gb1, gb2, hb0, hb1, hb2,
             xrem, grem, hrem, histg, histh, sem0, sem1, sem2, semr):
  c = lax.axis_index("c")
  s = lax.axis_index("s")
  wid = s * 2 + c
  base = wid * ROWS_PER_TILE

  zeros16 = jnp.zeros((16,), jnp.float32)

  @plsc.parallel_loop(0, HIST // 16, 1, unroll=8)
  def _zero(i):
    histg[pl.ds(i * 16, 16)] = zeros16
    histh[pl.ds(i * 16, 16)] = zeros16

  xbufs = (xb0, xb1, xb2)
  gbufs = (gb0, gb1, gb2)
  hbufs = (hb0, hb1, hb2)
  sems = (sem0, sem1, sem2)

  def start(ci, slot):
    st = base + ci * CHUNK
    pltpu.async_copy(x_hbm.at[pl.ds(st, CHUNK), :], xbufs[slot], sems[slot])
    pltpu.async_copy(g_hbm.at[pl.ds(st, CHUNK)],
                     gbufs[slot].at[pl.ds(0, CHUNK)], sems[slot])
    pltpu.async_copy(h_hbm.at[pl.ds(st, CHUNK)],
                     hbufs[slot].at[pl.ds(0, CHUNK)], sems[slot])

  def wait_slot(slot):
    pltpu.make_async_copy(x_hbm.at[pl.ds(0, CHUNK), :], xbufs[slot],
                          sems[slot]).wait()
    pltpu.make_async_copy(g_hbm.at[pl.ds(0, CHUNK)],
                          gbufs[slot].at[pl.ds(0, CHUNK)], sems[slot]).wait()
    pltpu.make_async_copy(h_hbm.at[pl.ds(0, CHUNK)],
                          hbufs[slot].at[pl.ds(0, CHUNK)], sems[slot]).wait()

  iota16 = lax.iota(jnp.int32, 16)
  feats = (iota16, iota16 + 16)

  def process(xb, gb, hb, nrows):
    @plsc.parallel_loop(0, nrows, 1, unroll=8)
    def _row(r):
      gv = jnp.full((16,), gb[pl.ds(r, 16)][0])
      hv = jnp.full((16,), hb[pl.ds(r, 16)][0])
      for half in range(2):
        xv = xb[r, pl.ds(half * 16, 16)]
        idx = (xv << 5) + feats[half]
        plsc.addupdate_scatter(histg, [idx], gv)
        plsc.addupdate_scatter(histh, [idx], hv)

  start(0, 0)
  start(1, 1)

  def outer(j, carry):
    for k in range(NBUF):
      ci = j * NBUF + k
      wait_slot(k)

      @pl.when(ci + 2 < NCHUNK)
      def _():
        start(ci + 2, (k + 2) % NBUF)

      process(xbufs[k], gbufs[k], hbufs[k], CHUNK)
    return carry
  lax.fori_loop(0, NOUTER, outer, 0)

  @pl.when(wid == 0)
  def _rem():
    st = NTILES * ROWS_PER_TILE
    pltpu.async_copy(x_hbm.at[pl.ds(st, REM), :], xrem, semr)
    pltpu.async_copy(g_hbm.at[pl.ds(st, REM)],
                     grem.at[pl.ds(0, REM)], semr)
    pltpu.async_copy(h_hbm.at[pl.ds(st, REM)],
                     hrem.at[pl.ds(0, REM)], semr)
    pltpu.make_async_copy(x_hbm.at[pl.ds(st, REM), :], xrem, semr).wait()
    pltpu.make_async_copy(g_hbm.at[pl.ds(st, REM)],
                          grem.at[pl.ds(0, REM)], semr).wait()
    pltpu.make_async_copy(h_hbm.at[pl.ds(st, REM)],
                          hrem.at[pl.ds(0, REM)], semr).wait()
    process(xrem, grem, hrem, REM)

  pltpu.sync_copy(histg, pg_hbm.at[pl.ds(wid * HIST, HIST)])
  pltpu.sync_copy(histh, ph_hbm.at[pl.ds(wid * HIST, HIST)])


def _tc_body(pg_ref, ph_ref, gl_ref, hl_ref):
  hg = jnp.sum(pg_ref[...], axis=0)
  hh = jnp.sum(ph_ref[...], axis=0)
  rows = lax.broadcasted_iota(jnp.int32, (NBIN, NBIN), 0)
  cols = lax.broadcasted_iota(jnp.int32, (NBIN, NBIN), 1)
  tri = (rows <= cols).astype(jnp.float32)
  dn = (((0,), (0,)), ((), ()))
  gl_ref[...] = lax.dot_general(hg, tri, dn,
                                preferred_element_type=jnp.float32)
  hl_ref[...] = lax.dot_general(hh, tri, dn,
                                preferred_element_type=jnp.float32)


@jax.jit
def kernel(X, gradient, hessian):
  mesh = plsc.VectorSubcoreMesh(core_axis_name="c", subcore_axis_name="s")
  sc = pl.kernel(
      _sc_body,
      out_type=(
          jax.ShapeDtypeStruct((NTILES * HIST,), jnp.float32),
          jax.ShapeDtypeStruct((NTILES * HIST,), jnp.float32),
      ),
      mesh=mesh,
      compiler_params=pltpu.CompilerParams(needs_layout_passes=False),
      scratch_types=[
          pltpu.VMEM((CHUNK, F), jnp.int32),
          pltpu.VMEM((CHUNK, F), jnp.int32),
          pltpu.VMEM((CHUNK, F), jnp.int32),
          pltpu.VMEM((GBUF,), jnp.float32),
          pltpu.VMEM((GBUF,), jnp.float32),
          pltpu.VMEM((GBUF,), jnp.float32),
          pltpu.VMEM((GBUF,), jnp.float32),
          pltpu.VMEM((GBUF,), jnp.float32),
          pltpu.VMEM((GBUF,), jnp.float32),
          pltpu.VMEM((REM, F), jnp.int32),
          pltpu.VMEM((REM + 16,), jnp.float32),
          pltpu.VMEM((REM + 16,), jnp.float32),
          pltpu.VMEM((HIST,), jnp.float32),
          pltpu.VMEM((HIST,), jnp.float32),
          pltpu.SemaphoreType.DMA,
          pltpu.SemaphoreType.DMA,
          pltpu.SemaphoreType.DMA,
          pltpu.SemaphoreType.DMA,
      ],
  )
  pg, ph = sc(X, gradient, hessian)

  pg3 = pg.reshape(NTILES, NBIN, F)
  ph3 = ph.reshape(NTILES, NBIN, F)
  gl, hl = pl.pallas_call(
      _tc_body,
      out_shape=(
          jax.ShapeDtypeStruct((F, NBIN), jnp.float32),
          jax.ShapeDtypeStruct((F, NBIN), jnp.float32),
      ),
  )(pg3, ph3)
  return (gl[None], hl[None])
